# baseline (speedup 1.0000x reference)
import jax
import jax.numpy as jnp
from jax.experimental import pallas as pl

N = 10000
E = 160000
D = 256
K = 2500


def _score_body(x_ref, sums_ref, deg_ref, ws_ref, wn_ref, b_ref, wsc_ref, bsc_ref,
                h_ref, score_ref):
    x = x_ref[...]
    nm = sums_ref[...] / deg_ref[...]
    h = jnp.dot(x, ws_ref[...]) + jnp.dot(nm, wn_ref[...]) + b_ref[...]
    h_ref[...] = h
    logit = jnp.dot(h, wsc_ref[...]) + bsc_ref[...]
    score_ref[...] = jax.nn.sigmoid(logit)


def kernel(x, edge_index, W_self, W_neigh, b, W_score, b_score, k):
    src = edge_index[0]
    dst = edge_index[1]
    msg = jnp.take(x, src, axis=0)
    sums = jax.ops.segment_sum(msg, dst, num_segments=N) + x
    deg = jax.ops.segment_sum(jnp.ones((E,), jnp.float32), dst, num_segments=N) + 1.0

    h, score = pl.pallas_call(
        _score_body,
        out_shape=(
            jax.ShapeDtypeStruct((N, D), jnp.float32),
            jax.ShapeDtypeStruct((N, 1), jnp.float32),
        ),
    )(x, sums, deg[:, None], W_self, W_neigh, b[None, :], W_score, b_score[None, :])

    sel_scores, idx = jax.lax.top_k(score[:, 0], K)
    idx = idx + (jnp.asarray(k, idx.dtype) - K)
    new_val = jnp.take(h, idx, axis=0) * sel_scores[:, None]
    return new_val, idx


# speedup vs baseline: 1.0042x; 1.0042x over previous
import jax
import jax.numpy as jnp
from jax import lax
from jax.experimental import pallas as pl
from jax.experimental.pallas import tpu as pltpu
from jax.experimental.pallas import tpu_sc as plsc

N = 10000
E = 160000
D = 256
K = 2500

KPT = 80
KPAD = 32 * KPT


def _score_body(x_ref, seg_ref, degc_ref, ws_ref, wn_ref, b_ref, wsc_ref, bsc_ref,
                h_ref, score_ref):
    x = x_ref[...]
    sums = seg_ref[...] + x
    deg = degc_ref[...] + 1.0
    nm = sums / deg
    h = jnp.dot(x, ws_ref[...]) + jnp.dot(nm, wn_ref[...]) + b_ref[...]
    h_ref[...] = h
    logit = jnp.dot(h, wsc_ref[...]) + bsc_ref[...]
    score_ref[...] = jax.nn.sigmoid(logit)


def _pool_body(h_hbm, idx_hbm, sc_hbm, out_hbm, ibuf, sbuf, iidx, stag, obuf, sem):
    sid = lax.axis_index("s")
    cid = lax.axis_index("c")
    wid = cid * 16 + sid
    base = wid * KPT

    pltpu.sync_copy(idx_hbm.at[pl.ds(base, KPT)], ibuf)
    pltpu.sync_copy(sc_hbm.at[pl.ds(base, KPT)], sbuf)

    def _group(i, _):
        iidx[0] = ibuf[pl.ds(16 * i, 16)]
        pltpu.sync_copy(h_hbm.at[iidx.at[0]], stag)
        s16 = sbuf[pl.ds(16 * i, 16)]
        for e in range(16):
            se = s16[e]
            for c in range(16):
                obuf[e, pl.ds(16 * c, 16)] = stag[e, pl.ds(16 * c, 16)] * se
        pltpu.sync_copy(obuf, out_hbm.at[pl.ds(base + 16 * i, 16)])
        return 0

    lax.fori_loop(0, KPT // 16, _group, 0)


def _pool(h, idx, sel_scores):
    idx_pad = jnp.concatenate([idx, jnp.zeros((KPAD - K,), jnp.int32)])
    sc_pad = jnp.concatenate([sel_scores, jnp.zeros((KPAD - K,), jnp.float32)])
    mesh = plsc.VectorSubcoreMesh(core_axis_name="c", subcore_axis_name="s")
    f = pl.kernel(
        _pool_body,
        out_type=jax.ShapeDtypeStruct((KPAD, D), jnp.float32),
        mesh=mesh,
        scratch_types=[
            pltpu.VMEM((KPT,), jnp.int32),
            pltpu.VMEM((KPT,), jnp.float32),
            pltpu.VMEM((1, 16), jnp.int32),
            pltpu.VMEM((16, D), jnp.float32),
            pltpu.VMEM((16, D), jnp.float32),
            pltpu.SemaphoreType.DMA,
        ],
    )
    return f(h, idx_pad, sc_pad)[:K]


def kernel(x, edge_index, W_self, W_neigh, b, W_score, b_score, k):
    src = edge_index[0]
    dst = edge_index[1]
    msg = jnp.take(x, src, axis=0)
    seg = jax.ops.segment_sum(msg, dst, num_segments=N)
    degc = jax.ops.segment_sum(jnp.ones((E,), jnp.float32), dst, num_segments=N)

    h, score = pl.pallas_call(
        _score_body,
        out_shape=(
            jax.ShapeDtypeStruct((N, D), jnp.float32),
            jax.ShapeDtypeStruct((N, 1), jnp.float32),
        ),
    )(x, seg, degc[:, None], W_self, W_neigh, b[None, :], W_score, b_score[None, :])

    sel_scores, idx = jax.lax.top_k(score[:, 0], K)
    idx = idx + (jnp.asarray(k, idx.dtype) - K)
    new_val = _pool(h, idx, sel_scores)
    return new_val, idx
